# baseline (device time: 206122 ns/iter reference)
import jax
import jax.numpy as jnp
from jax import lax
from jax.experimental import pallas as pl
from jax.experimental.pallas import tpu as pltpu

N_DEV = 8
SQ = 512
SKV = 2048
D_MODEL = 1024
HQ_PER = 8
DH = 128
SCALE = 0.08838834764831843


def _attention(head0, xb, Wq, K2, V2):

    def body(h0_ref, x_ref, wq_ref, k_ref, v_ref, o_ref):
        q = jnp.dot(x_ref[...], wq_ref[...],
                    preferred_element_type=jnp.float32) * SCALE
        q = q.astype(jnp.bfloat16)
        k = k_ref[...].astype(jnp.bfloat16)
        v = v_ref[...].astype(jnp.bfloat16)
        s = lax.dot_general(q, k, (((1,), (1,)), ((), ())),
                            preferred_element_type=jnp.float32)
        p = jnp.exp(s)
        l = jnp.sum(p, axis=-1, keepdims=True)
        o = jnp.dot(p.astype(jnp.bfloat16), v,
                    preferred_element_type=jnp.float32) / l
        o_ref[...] = o.astype(jnp.bfloat16)

    grid_spec = pltpu.PrefetchScalarGridSpec(
        num_scalar_prefetch=1,
        grid=(HQ_PER,),
        in_specs=[
            pl.BlockSpec((SQ, D_MODEL), lambda h, s: (0, 0)),
            pl.BlockSpec((D_MODEL, DH), lambda h, s: (0, h)),
            pl.BlockSpec((SKV, DH), lambda h, s: (0, s[0] + h)),
            pl.BlockSpec((SKV, DH), lambda h, s: (0, s[0] + h)),
        ],
        out_specs=pl.BlockSpec((SQ, DH), lambda h, s: (0, h)),
    )
    return pl.pallas_call(
        body,
        grid_spec=grid_spec,
        out_shape=jax.ShapeDtypeStruct((SQ, HQ_PER * DH), jnp.bfloat16),
    )(head0, xb, Wq, K2, V2)


def _project_allreduce(o, Wo):

    def body(o_ref, wo_ref, out_ref, stage_ref, xstage_ref, recv_ref,
             send_sems, recv_sems):
        my = lax.axis_index("i")
        r1 = (my >> 1) & 1
        r2 = my & 1
        q1 = my ^ 3
        q2 = my ^ 1
        q3 = my ^ 4

        barrier_sem = pltpu.get_barrier_semaphore()
        for nbr in (q1, q2, q3):
            pl.semaphore_signal(
                barrier_sem, inc=1,
                device_id=(nbr,), device_id_type=pl.DeviceIdType.MESH,
            )
        pl.semaphore_wait(barrier_sem, 3)

        out_ref[...] = jnp.dot(o_ref[...], wo_ref[...],
                               preferred_element_type=jnp.float32)

        kept1_lo = r1 * 256
        sent1_lo = (1 - r1) * 256
        kept2_lo = kept1_lo + r2 * 128
        sent2_lo = kept1_lo + (1 - r2) * 128

        pending = []

        def exchange(idx, partner, src_ref, src_lo, dst_ref, dst_lo, ln):
            rdma = pltpu.make_async_remote_copy(
                src_ref=src_ref.at[pl.ds(src_lo, ln), :],
                dst_ref=dst_ref.at[pl.ds(dst_lo, ln), :],
                send_sem=send_sems.at[idx],
                recv_sem=recv_sems.at[idx],
                device_id=(partner,),
                device_id_type=pl.DeviceIdType.MESH,
            )
            rdma.start()
            rdma.wait_recv()
            pending.append(rdma)

        stage_ref[pl.ds(sent1_lo, 256), :] = (
            out_ref[pl.ds(sent1_lo, 256), :].astype(jnp.bfloat16))
        exchange(0, q1, stage_ref, sent1_lo, recv_ref, 0, 256)
        out_ref[pl.ds(kept1_lo, 256), :] += (
            recv_ref[pl.ds(0, 256), :].astype(jnp.float32))

        stage_ref[pl.ds(sent2_lo, 128), :] = (
            out_ref[pl.ds(sent2_lo, 128), :].astype(jnp.bfloat16))
        exchange(1, q2, stage_ref, sent2_lo, recv_ref, 256, 128)
        out_ref[pl.ds(kept2_lo, 128), :] += (
            recv_ref[pl.ds(256, 128), :].astype(jnp.float32))

        xstage_ref[...] = out_ref[pl.ds(kept2_lo, 128), :].astype(jnp.bfloat16)
        exchange(2, q3, xstage_ref, 0, recv_ref, 384, 128)
        out_ref[pl.ds(kept2_lo, 128), :] += (
            recv_ref[pl.ds(384, 128), :].astype(jnp.float32))

        stage_ref[pl.ds(kept2_lo, 128), :] = (
            out_ref[pl.ds(kept2_lo, 128), :].astype(jnp.bfloat16))
        exchange(3, q2, stage_ref, kept2_lo, stage_ref, kept2_lo, 128)

        exchange(4, q1, stage_ref, kept1_lo, stage_ref, kept1_lo, 256)

        out_ref[...] = stage_ref[...].astype(jnp.float32)

        for rdma in pending:
            rdma.wait_send()

    return pl.pallas_call(
        body,
        out_shape=jax.ShapeDtypeStruct((SQ, D_MODEL), jnp.float32),
        in_specs=[
            pl.BlockSpec(memory_space=pltpu.VMEM),
            pl.BlockSpec(memory_space=pltpu.VMEM),
        ],
        out_specs=pl.BlockSpec(memory_space=pltpu.VMEM),
        scratch_shapes=[
            pltpu.VMEM((SQ, D_MODEL), jnp.bfloat16),
            pltpu.VMEM((128, D_MODEL), jnp.bfloat16),
            pltpu.VMEM((SQ, D_MODEL), jnp.bfloat16),
            pltpu.SemaphoreType.DMA((5,)),
            pltpu.SemaphoreType.DMA((5,)),
        ],
        compiler_params=pltpu.CompilerParams(collective_id=0),
    )(o, Wo)


def kernel(x, Wq, Wo, K_ext, V_ext):
    my = lax.axis_index("i")

    xb = x[0].astype(jnp.bfloat16)
    K2 = K_ext.reshape(SKV, 64 * DH)
    V2 = V_ext.reshape(SKV, 64 * DH)
    head0 = jnp.reshape(my * HQ_PER, (1,)).astype(jnp.int32)

    o = _attention(head0, xb, Wq.astype(jnp.bfloat16), K2, V2)
    out = _project_allreduce(o, Wo.astype(jnp.bfloat16))
    return out.reshape(1, SQ, D_MODEL)


# device time: 68240 ns/iter; 3.0205x vs baseline; 3.0205x over previous
import jax
import jax.numpy as jnp
from jax import lax
from jax.experimental import pallas as pl
from jax.experimental.pallas import tpu as pltpu

N_DEV = 8
SQ = 512
SKV = 2048
D_MODEL = 1024
HQ_PER = 8
DH = 128
SCALE = 0.08838834764831843


def _attention(xb, Wq, K, V):

    def body(x_ref, wq_ref, k_ref, v_ref, o_ref):
        q = jnp.dot(x_ref[...], wq_ref[...],
                    preferred_element_type=jnp.float32) * SCALE
        q = q.astype(jnp.bfloat16)
        k = k_ref[0]
        v = v_ref[0]
        s = lax.dot_general(q, k, (((1,), (1,)), ((), ())),
                            preferred_element_type=jnp.float32)
        p = jnp.exp(s)
        l = jnp.sum(p, axis=-1, keepdims=True)
        o = jnp.dot(p.astype(jnp.bfloat16), v,
                    preferred_element_type=jnp.float32) / l
        o_ref[...] = o.astype(jnp.bfloat16)

    return pl.pallas_call(
        body,
        grid=(HQ_PER,),
        in_specs=[
            pl.BlockSpec((SQ, D_MODEL), lambda h: (0, 0)),
            pl.BlockSpec((D_MODEL, DH), lambda h: (0, h)),
            pl.BlockSpec((1, SKV, DH), lambda h: (h, 0, 0)),
            pl.BlockSpec((1, SKV, DH), lambda h: (h, 0, 0)),
        ],
        out_specs=pl.BlockSpec((SQ, DH), lambda h: (0, h)),
        out_shape=jax.ShapeDtypeStruct((SQ, HQ_PER * DH), jnp.bfloat16),
    )(xb, Wq, K, V)


def _project_allreduce(o, Wo):

    def body(o_ref, wo_ref, out_ref, stage_ref, xstage_ref, recv_ref,
             send_sems, recv_sems):
        my = lax.axis_index("i")
        r1 = (my >> 1) & 1
        r2 = my & 1
        q1 = my ^ 3
        q2 = my ^ 1
        q3 = my ^ 4

        barrier_sem = pltpu.get_barrier_semaphore()
        for nbr in (q1, q2, q3):
            pl.semaphore_signal(
                barrier_sem, inc=1,
                device_id=(nbr,), device_id_type=pl.DeviceIdType.MESH,
            )
        pl.semaphore_wait(barrier_sem, 3)

        out_ref[...] = jnp.dot(o_ref[...], wo_ref[...],
                               preferred_element_type=jnp.float32)

        kept1_lo = r1 * 256
        sent1_lo = (1 - r1) * 256
        kept2_lo = kept1_lo + r2 * 128
        sent2_lo = kept1_lo + (1 - r2) * 128

        pending = []

        def exchange(idx, partner, src_ref, src_lo, dst_ref, dst_lo, ln):
            rdma = pltpu.make_async_remote_copy(
                src_ref=src_ref.at[pl.ds(src_lo, ln), :],
                dst_ref=dst_ref.at[pl.ds(dst_lo, ln), :],
                send_sem=send_sems.at[idx],
                recv_sem=recv_sems.at[idx],
                device_id=(partner,),
                device_id_type=pl.DeviceIdType.MESH,
            )
            rdma.start()
            rdma.wait_recv()
            pending.append(rdma)

        stage_ref[pl.ds(sent1_lo, 256), :] = (
            out_ref[pl.ds(sent1_lo, 256), :].astype(jnp.bfloat16))
        exchange(0, q1, stage_ref, sent1_lo, recv_ref, 0, 256)
        out_ref[pl.ds(kept1_lo, 256), :] += (
            recv_ref[pl.ds(0, 256), :].astype(jnp.float32))

        stage_ref[pl.ds(sent2_lo, 128), :] = (
            out_ref[pl.ds(sent2_lo, 128), :].astype(jnp.bfloat16))
        exchange(1, q2, stage_ref, sent2_lo, recv_ref, 256, 128)
        out_ref[pl.ds(kept2_lo, 128), :] += (
            recv_ref[pl.ds(256, 128), :].astype(jnp.float32))

        xstage_ref[...] = out_ref[pl.ds(kept2_lo, 128), :].astype(jnp.bfloat16)
        exchange(2, q3, xstage_ref, 0, recv_ref, 384, 128)
        out_ref[pl.ds(kept2_lo, 128), :] += (
            recv_ref[pl.ds(384, 128), :].astype(jnp.float32))

        stage_ref[pl.ds(kept2_lo, 128), :] = (
            out_ref[pl.ds(kept2_lo, 128), :].astype(jnp.bfloat16))
        exchange(3, q2, stage_ref, kept2_lo, stage_ref, kept2_lo, 128)

        exchange(4, q1, stage_ref, kept1_lo, stage_ref, kept1_lo, 256)

        out_ref[...] = stage_ref[...].astype(jnp.float32)

        for rdma in pending:
            rdma.wait_send()

    return pl.pallas_call(
        body,
        out_shape=jax.ShapeDtypeStruct((SQ, D_MODEL), jnp.float32),
        in_specs=[
            pl.BlockSpec(memory_space=pltpu.VMEM),
            pl.BlockSpec(memory_space=pltpu.VMEM),
        ],
        out_specs=pl.BlockSpec(memory_space=pltpu.VMEM),
        scratch_shapes=[
            pltpu.VMEM((SQ, D_MODEL), jnp.bfloat16),
            pltpu.VMEM((128, D_MODEL), jnp.bfloat16),
            pltpu.VMEM((SQ, D_MODEL), jnp.bfloat16),
            pltpu.SemaphoreType.DMA((5,)),
            pltpu.SemaphoreType.DMA((5,)),
        ],
        compiler_params=pltpu.CompilerParams(collective_id=0),
    )(o, Wo)


def kernel(x, Wq, Wo, K_ext, V_ext):
    my = lax.axis_index("i")

    xb = x[0].astype(jnp.bfloat16)
    K = lax.dynamic_slice_in_dim(K_ext[0], my * HQ_PER, HQ_PER, axis=1)
    V = lax.dynamic_slice_in_dim(V_ext[0], my * HQ_PER, HQ_PER, axis=1)
    K = K.astype(jnp.bfloat16).transpose(1, 0, 2)
    V = V.astype(jnp.bfloat16).transpose(1, 0, 2)

    o = _attention(xb, Wq.astype(jnp.bfloat16), K, V)
    out = _project_allreduce(o, Wo.astype(jnp.bfloat16))
    return out.reshape(1, SQ, D_MODEL)


# device time: 57584 ns/iter; 3.5795x vs baseline; 1.1851x over previous
import jax
import jax.numpy as jnp
from jax import lax
from jax.experimental import pallas as pl
from jax.experimental.pallas import tpu as pltpu

N_DEV = 8
SQ = 512
SKV = 2048
D_MODEL = 1024
HQ_PER = 8
DH = 128
SCALE = 0.08838834764831843


def _attention(xb, Wq, K, V):

    def body(x_ref, wq_ref, k_ref, v_ref, o_ref):
        q = jnp.dot(x_ref[...], wq_ref[...],
                    preferred_element_type=jnp.float32) * SCALE
        q = q.astype(jnp.bfloat16)
        k = k_ref[0]
        v = v_ref[0]
        s = lax.dot_general(q, k, (((1,), (1,)), ((), ())),
                            preferred_element_type=jnp.float32)
        p = jnp.exp(s)
        l = jnp.sum(p, axis=-1, keepdims=True)
        o = jnp.dot(p.astype(jnp.bfloat16), v,
                    preferred_element_type=jnp.float32) / l
        o_ref[...] = o.astype(jnp.bfloat16)

    return pl.pallas_call(
        body,
        grid=(HQ_PER,),
        in_specs=[
            pl.BlockSpec((SQ, D_MODEL), lambda h: (0, 0)),
            pl.BlockSpec((D_MODEL, DH), lambda h: (0, h)),
            pl.BlockSpec((1, SKV, DH), lambda h: (h, 0, 0)),
            pl.BlockSpec((1, SKV, DH), lambda h: (h, 0, 0)),
        ],
        out_specs=pl.BlockSpec((SQ, DH), lambda h: (0, h)),
        out_shape=jax.ShapeDtypeStruct((SQ, HQ_PER * DH), jnp.bfloat16),
    )(xb, Wq, K, V)


ROWS_PER = SQ // N_DEV


def _project_allreduce(o, Wo):

    def body(o_ref, wo_ref, out_ref, stage_ref, rs_recv_ref, gather_ref,
             rs_send_sems, rs_recv_sems, ag_send_sems, ag_recv_sems):
        my = lax.axis_index("i")

        barrier_sem = pltpu.get_barrier_semaphore()
        for p in range(N_DEV):
            @pl.when(p != my)
            def _():
                pl.semaphore_signal(
                    barrier_sem, inc=1,
                    device_id=(p,), device_id_type=pl.DeviceIdType.MESH,
                )
        pl.semaphore_wait(barrier_sem, N_DEV - 1)

        out_ref[...] = jnp.dot(o_ref[...], wo_ref[...],
                               preferred_element_type=jnp.float32)
        stage_ref[...] = out_ref[...].astype(jnp.bfloat16)

        for p in range(N_DEV):
            @pl.when(p != my)
            def _():
                rdma = pltpu.make_async_remote_copy(
                    src_ref=stage_ref.at[pl.ds(p * ROWS_PER, ROWS_PER), :],
                    dst_ref=rs_recv_ref.at[my],
                    send_sem=rs_send_sems.at[p],
                    recv_sem=rs_recv_sems.at[my],
                    device_id=(p,),
                    device_id_type=pl.DeviceIdType.MESH,
                )
                rdma.start()

        for s in range(N_DEV):
            @pl.when(s != my)
            def _():
                recv = pltpu.make_async_remote_copy(
                    src_ref=stage_ref.at[pl.ds(0, ROWS_PER), :],
                    dst_ref=rs_recv_ref.at[s],
                    send_sem=rs_send_sems.at[s],
                    recv_sem=rs_recv_sems.at[s],
                    device_id=(s,),
                    device_id_type=pl.DeviceIdType.MESH,
                )
                recv.wait_recv()
                out_ref[pl.ds(my * ROWS_PER, ROWS_PER), :] += (
                    rs_recv_ref[s].astype(jnp.float32))

        stage_ref[pl.ds(my * ROWS_PER, ROWS_PER), :] = (
            out_ref[pl.ds(my * ROWS_PER, ROWS_PER), :].astype(jnp.bfloat16))
        gather_ref[pl.ds(my * ROWS_PER, ROWS_PER), :] = (
            stage_ref[pl.ds(my * ROWS_PER, ROWS_PER), :])
        for p in range(N_DEV):
            @pl.when(p != my)
            def _():
                rdma = pltpu.make_async_remote_copy(
                    src_ref=stage_ref.at[pl.ds(my * ROWS_PER, ROWS_PER), :],
                    dst_ref=gather_ref.at[pl.ds(my * ROWS_PER, ROWS_PER), :],
                    send_sem=ag_send_sems.at[p],
                    recv_sem=ag_recv_sems.at[my],
                    device_id=(p,),
                    device_id_type=pl.DeviceIdType.MESH,
                )
                rdma.start()

        for s in range(N_DEV):
            @pl.when(s != my)
            def _():
                recv = pltpu.make_async_remote_copy(
                    src_ref=stage_ref.at[pl.ds(0, ROWS_PER), :],
                    dst_ref=gather_ref.at[pl.ds(s * ROWS_PER, ROWS_PER), :],
                    send_sem=ag_send_sems.at[s],
                    recv_sem=ag_recv_sems.at[s],
                    device_id=(s,),
                    device_id_type=pl.DeviceIdType.MESH,
                )
                recv.wait_recv()

        out_ref[...] = gather_ref[...].astype(jnp.float32)

        for p in range(N_DEV):
            @pl.when(p != my)
            def _():
                for sems in (rs_send_sems, ag_send_sems):
                    drain = pltpu.make_async_remote_copy(
                        src_ref=stage_ref.at[pl.ds(0, ROWS_PER), :],
                        dst_ref=rs_recv_ref.at[0],
                        send_sem=sems.at[p],
                        recv_sem=rs_recv_sems.at[0],
                        device_id=(p,),
                        device_id_type=pl.DeviceIdType.MESH,
                    )
                    drain.wait_send()

    return pl.pallas_call(
        body,
        out_shape=jax.ShapeDtypeStruct((SQ, D_MODEL), jnp.float32),
        in_specs=[
            pl.BlockSpec(memory_space=pltpu.VMEM),
            pl.BlockSpec(memory_space=pltpu.VMEM),
        ],
        out_specs=pl.BlockSpec(memory_space=pltpu.VMEM),
        scratch_shapes=[
            pltpu.VMEM((SQ, D_MODEL), jnp.bfloat16),
            pltpu.VMEM((N_DEV, ROWS_PER, D_MODEL), jnp.bfloat16),
            pltpu.VMEM((SQ, D_MODEL), jnp.bfloat16),
            pltpu.SemaphoreType.DMA((N_DEV,)),
            pltpu.SemaphoreType.DMA((N_DEV,)),
            pltpu.SemaphoreType.DMA((N_DEV,)),
            pltpu.SemaphoreType.DMA((N_DEV,)),
        ],
        compiler_params=pltpu.CompilerParams(collective_id=0),
    )(o, Wo)


def kernel(x, Wq, Wo, K_ext, V_ext):
    my = lax.axis_index("i")

    xb = x[0].astype(jnp.bfloat16)
    K = lax.dynamic_slice_in_dim(K_ext[0], my * HQ_PER, HQ_PER, axis=1)
    V = lax.dynamic_slice_in_dim(V_ext[0], my * HQ_PER, HQ_PER, axis=1)
    K = K.astype(jnp.bfloat16).transpose(1, 0, 2)
    V = V.astype(jnp.bfloat16).transpose(1, 0, 2)

    o = _attention(xb, Wq.astype(jnp.bfloat16), K, V)
    out = _project_allreduce(o, Wo.astype(jnp.bfloat16))
    return out.reshape(1, SQ, D_MODEL)
